# Initial kernel scaffold; baseline (speedup 1.0000x reference)
import jax
import jax.numpy as jnp
from jax.experimental import pallas as pl


def kernel(prp_boxes, prp_labels, tgt_boxes, tgt_labels, tgt_rel_matrix):
    raise NotImplementedError("write your pallas kernel here")


# speedup vs baseline: 1.0246x; 1.0246x over previous
import jax
import jax.numpy as jnp
from jax import lax
from jax.experimental import pallas as pl
from jax.experimental.pallas import tpu as pltpu

_P = 512
_T = 64
_NUM_POS = 256
_NUM_NEG = 768
_FG_THRES = 0.5


def _dense_body(tb_ref, pbt_ref, tl_ref, plr_ref, plc_ref, rel_ref,
                fg_scores_ref, bgmask_ref, binary_ref, m_ref, innerlab_ref):
    tb = tb_ref[...]
    pbt = pbt_ref[...]
    tl = tl_ref[...]
    plr = plr_ref[...]
    rel = rel_ref[...].astype(jnp.float32)

    tx0 = tb[:, 0:1]
    ty0 = tb[:, 1:2]
    tx1 = tb[:, 2:3]
    ty1 = tb[:, 3:4]
    px0 = pbt[0:1, :]
    py0 = pbt[1:2, :]
    px1 = pbt[2:3, :]
    py1 = pbt[3:4, :]

    area_t = (tx1 - tx0) * (ty1 - ty0)
    area_p = (px1 - px0) * (py1 - py0)
    w = jnp.clip(jnp.minimum(tx1, px1) - jnp.maximum(tx0, px0), 0.0)
    h = jnp.clip(jnp.minimum(ty1, py1) - jnp.maximum(ty0, py0), 0.0)
    inter = w * h
    union = area_t + area_p - inter
    ious = inter / jnp.maximum(union, 1e-8)

    m = ((tl == plr) & (ious > _FG_THRES)).astype(jnp.float32)
    relpos = (rel > 0).astype(jnp.float32)

    inner_lab = jnp.zeros((_T, _P), jnp.float32)
    inner_iou = jnp.zeros((_T, _P), jnp.float32)
    for t in range(_T):
        inner_lab = jnp.maximum(inner_lab, rel[:, t:t + 1] * m[t:t + 1, :])
        inner_iou = jnp.maximum(inner_iou, relpos[:, t:t + 1] * ious[t:t + 1, :])

    t1 = jnp.dot(relpos, m, preferred_element_type=jnp.float32)
    b3 = lax.dot_general(m, t1, (((0,), (0,)), ((), ())),
                         preferred_element_type=jnp.float32)
    rows = lax.broadcasted_iota(jnp.int32, (_P, _P), 0)
    cols = lax.broadcasted_iota(jnp.int32, (_P, _P), 1)
    offdiag = rows != cols
    fg_mask = (b3 > 0) & offdiag

    binary_ref[...] = ((b3 + b3.T) > 0).astype(jnp.int32)

    ioust = ious.T
    iou_pair = jnp.zeros((_P, _P), jnp.float32)
    for hh in range(_T):
        iou_pair = jnp.maximum(iou_pair, ioust[:, hh:hh + 1] * inner_iou[hh:hh + 1, :])

    fg_scores_ref[...] = jnp.where(fg_mask, iou_pair, 0.0)

    validr = (plr != 0)
    validc = (plc_ref[...] != 0)
    bgmask_ref[...] = (validc & validr & offdiag & jnp.logical_not(fg_mask)
                       ).astype(jnp.float32)

    m_ref[...] = m
    innerlab_ref[...] = inner_lab


def _dense_call(tgt_boxes, prp_boxes, tgt_labels, prp_labels, tgt_rel_matrix):
    out_shapes = (
        jax.ShapeDtypeStruct((_P, _P), jnp.float32),
        jax.ShapeDtypeStruct((_P, _P), jnp.float32),
        jax.ShapeDtypeStruct((_P, _P), jnp.int32),
        jax.ShapeDtypeStruct((_T, _P), jnp.float32),
        jax.ShapeDtypeStruct((_T, _P), jnp.float32),
    )
    return pl.pallas_call(
        _dense_body,
        out_shape=out_shapes,
    )(tgt_boxes.astype(jnp.float32),
      prp_boxes.astype(jnp.float32).T,
      tgt_labels.astype(jnp.int32).reshape(_T, 1),
      prp_labels.astype(jnp.int32).reshape(1, _P),
      prp_labels.astype(jnp.int32).reshape(_P, 1),
      tgt_rel_matrix.astype(jnp.int32))


def kernel(prp_boxes, prp_labels, tgt_boxes, tgt_labels, tgt_rel_matrix):
    fg_scores, bgmask, binary_rel, m, inner_lab = _dense_call(
        tgt_boxes, prp_boxes, tgt_labels, prp_labels, tgt_rel_matrix)

    fg_vals, fg_idx = lax.top_k(fg_scores.reshape(-1), _NUM_POS)
    fg_head = fg_idx // _P
    fg_tail = fg_idx % _P
    fg_valid = fg_vals > 0
    lab = jnp.max(m[:, fg_head] * inner_lab[:, fg_tail], axis=0)
    fg_labels = lab.astype(jnp.int32) * fg_valid.astype(jnp.int32)

    noise = jax.random.uniform(jax.random.key(42), (_P * _P,))
    bg_scores = jnp.where(bgmask.reshape(-1) > 0, noise, -1.0)
    _, bg_idx = lax.top_k(bg_scores, _NUM_NEG)
    bg_head = bg_idx // _P
    bg_tail = bg_idx % _P

    rel_pairs = jnp.stack(
        [jnp.concatenate([fg_head, bg_head]),
         jnp.concatenate([fg_tail, bg_tail])], axis=1)
    rel_labels = jnp.concatenate(
        [fg_labels, jnp.zeros((_NUM_NEG,), dtype=jnp.int32)])
    fg_quality = fg_vals * fg_valid.astype(fg_vals.dtype)
    return rel_pairs, rel_labels, binary_rel, fg_quality


# speedup vs baseline: 9.1236x; 8.9049x over previous
import numpy as np
import jax
import jax.numpy as jnp
from jax import lax
from jax.experimental import pallas as pl
from jax.experimental.pallas import tpu as pltpu

_P = 512
_T = 64
_NUM_POS = 256
_NUM_NEG = 768
_FG_THRES = 0.5
_HI = lax.Precision.HIGHEST
_BIG = 1 << 19

def _rank_matrix():
    with jax.ensure_compile_time_eval():
        noise = jax.random.uniform(jax.random.key(42), (_P * _P,))
        order = jnp.argsort(-noise, stable=True)
        inv = jnp.zeros((_P * _P,), jnp.int32).at[order].set(
            jnp.arange(_P * _P, dtype=jnp.int32))
        return np.asarray(inv).reshape(_P, _P)


_RR_CONST = _rank_matrix()


def _main_body(tb_ref, pbt_ref, tl_ref, plr_ref, plc_ref, rel_ref, rr_ref,
               fgidx_ref, fgval_ref, fglab_ref, bgidx_ref, binary_ref):
    f32 = jnp.float32
    i32 = jnp.int32
    ones_1p = jnp.ones((1, _P), f32)
    ones_p1 = jnp.ones((_P, 1), f32)
    i0 = lax.broadcasted_iota(i32, (_P, _P), 0)
    i1 = lax.broadcasted_iota(i32, (_P, _P), 1)
    ut = (i0 <= i1).astype(f32)
    identp = (i0 == i1).astype(f32)
    offdiag = i0 != i1
    jmat = i0 * _P + i1

    tb = tb_ref[...]
    pbt = pbt_ref[...]
    tl = tl_ref[...]
    plr = plr_ref[...]
    rel = rel_ref[...].astype(f32)

    tx0, ty0, tx1, ty1 = (tb[:, k:k + 1] for k in range(4))
    px0, py0, px1, py1 = (pbt[k:k + 1, :] for k in range(4))
    area_t = (tx1 - tx0) * (ty1 - ty0)
    area_p = (px1 - px0) * (py1 - py0)
    w = jnp.clip(jnp.minimum(tx1, px1) - jnp.maximum(tx0, px0), 0.0)
    h = jnp.clip(jnp.minimum(ty1, py1) - jnp.maximum(ty0, py0), 0.0)
    inter = w * h
    union = area_t + area_p - inter
    ious = inter / jnp.maximum(union, 1e-8)

    m = ((tl == plr) & (ious > _FG_THRES)).astype(f32)
    relpos = (rel > 0).astype(f32)

    inner_lab = jnp.zeros((_T, _P), f32)
    inner_iou = jnp.zeros((_T, _P), f32)
    for t in range(_T):
        inner_lab = jnp.maximum(inner_lab, rel[:, t:t + 1] * m[t:t + 1, :])
        inner_iou = jnp.maximum(inner_iou, relpos[:, t:t + 1] * ious[t:t + 1, :])

    t1 = jnp.dot(relpos, m, preferred_element_type=f32)
    b3 = lax.dot_general(m, t1, (((0,), (0,)), ((), ())),
                         preferred_element_type=f32)
    b3t = lax.dot_general(t1, m, (((0,), (0,)), ((), ())),
                          preferred_element_type=f32)
    fg_mask = (b3 > 0) & offdiag
    binary_ref[...] = ((b3 + b3t) > 0).astype(i32)

    ioust = ious.T
    iou_pair = jnp.zeros((_P, _P), f32)
    for hh in range(_T):
        iou_pair = jnp.maximum(iou_pair,
                               ioust[:, hh:hh + 1] * inner_iou[hh:hh + 1, :])

    s = jnp.where(fg_mask, iou_pair, 0.0)
    validr = plr != 0
    validc = plc_ref[...] != 0
    bgm = validc & validr & offdiag & jnp.logical_not(fg_mask)

    def prefix_struct(self_f):
        rowp = jnp.dot(self_f, ut, preferred_element_type=f32)
        rowcnt_row = lax.dot_general(ones_1p, self_f, (((1,), (1,)), ((), ())),
                                     preferred_element_type=f32)
        crow_row = jnp.dot(rowcnt_row, ut, preferred_element_type=f32, precision=_HI)
        return rowp, rowcnt_row, crow_row

    def enumerate_sel(nslots, rowp, rowcnt_row, crow_row):
        crow_excl_row = crow_row - rowcnt_row
        svec = lax.broadcasted_iota(i32, (nslots, 1), 0).astype(f32) + 1.0
        m1 = (crow_row < svec).astype(f32)
        r_col = jnp.sum(m1, axis=1, keepdims=True)
        lane = lax.broadcasted_iota(i32, (nslots, _P), 1)
        o_r = (lane == r_col.astype(i32)).astype(f32)
        ce_at = lax.dot_general(o_r, crow_excl_row, (((1,), (1,)), ((), ())),
                                preferred_element_type=f32, precision=_HI)
        tgt = svec - ce_at
        pref_rows = jnp.dot(o_r, rowp, preferred_element_type=f32, precision=_HI)
        q_col = jnp.sum((pref_rows < tgt).astype(f32), axis=1, keepdims=True)
        o_q = (lane == q_col.astype(i32)).astype(f32)
        j_col = r_col * float(_P) + q_col
        return j_col, o_r, o_q

    def gather_at(o_r, o_q, mat):
        rows = jnp.dot(o_r, mat, preferred_element_type=f32, precision=_HI)
        return jnp.sum(rows * o_q, axis=1, keepdims=True)

    bits = lax.bitcast_convert_type(s, i32)
    npos = jnp.sum((bits > 0).astype(i32))

    def search_fg(_):
        def bs(_, st):
            lo, hi = st
            mid = lo + (hi - lo) // 2
            pred = jnp.sum((bits > mid).astype(i32)) < _NUM_POS
            return (jnp.where(pred, lo, mid), jnp.where(pred, mid, hi))
        lo, hi = lax.fori_loop(0, 31, bs, (jnp.int32(-1), jnp.int32(0x3F800000)))
        return hi

    vstar = lax.cond(npos <= _NUM_POS, lambda _: jnp.int32(0), search_fg, None)
    gtf = (bits > vstar).astype(f32)
    eqf = (bits == vstar).astype(f32)
    n_gt = jnp.sum(gtf)
    need_eq = jnp.float32(_NUM_POS) - n_gt
    rowp_e, rowcnt_e, crow_e = prefix_struct(eqf)
    crow_e_excl_row = crow_e - rowcnt_e
    d_e = identp * crow_e_excl_row
    crow_e_excl_col = jnp.dot(d_e, ones_p1, preferred_element_type=f32, precision=_HI)
    excl_e = rowp_e - eqf + crow_e_excl_col
    self_fg = gtf + eqf * (excl_e < need_eq).astype(f32)

    rowp_f, rowcnt_f, crow_f = prefix_struct(self_fg)
    j_fg, o_r, o_q = enumerate_sel(_NUM_POS, rowp_f, rowcnt_f, crow_f)
    val_fg = gather_at(o_r, o_q, s)
    m_cols = lax.dot_general(o_r, m, (((1,), (1,)), ((), ())),
                             preferred_element_type=f32)
    il_cols = lax.dot_general(o_q, inner_lab, (((1,), (1,)), ((), ())),
                              preferred_element_type=f32)
    lab_fg = jnp.max(m_cols * il_cols, axis=1, keepdims=True)
    lab_fg = jnp.where(val_fg > 0, lab_fg, 0.0)

    ident_k = (lax.broadcasted_iota(i32, (_NUM_POS, _NUM_POS), 0) ==
               lax.broadcasted_iota(i32, (_NUM_POS, _NUM_POS), 1)).astype(f32)
    d_v = ident_k * val_fg
    vrow = jnp.dot(jnp.ones((1, _NUM_POS), f32), d_v,
                   preferred_element_type=f32, precision=_HI)
    kc = lax.broadcasted_iota(i32, (_NUM_POS, 1), 0)
    kr = lax.broadcasted_iota(i32, (1, _NUM_POS), 1)
    beats = (vrow > val_fg) | ((vrow == val_fg) & (kr < kc))
    rank_fg = jnp.sum(beats.astype(f32), axis=1, keepdims=True)
    operm = (lax.broadcasted_iota(i32, (_NUM_POS, _NUM_POS), 1) ==
             rank_fg.astype(i32)).astype(f32)

    def permute(operm_m, col):
        return lax.dot_general(operm_m, col, (((0,), (0,)), ((), ())),
                               preferred_element_type=f32, precision=_HI)

    fgval_ref[...] = permute(operm, val_fg)
    fgidx_ref[...] = permute(operm, j_fg).astype(i32)
    fglab_ref[...] = permute(operm, lab_fg).astype(i32)

    keyb = jnp.where(bgm, rr_ref[...], _BIG + jmat)

    def bs2(_, st):
        lo, hi = st
        mid = lo + (hi - lo) // 2
        pred = jnp.sum((keyb <= mid).astype(i32)) >= _NUM_NEG
        return (jnp.where(pred, lo, mid), jnp.where(pred, mid, hi))

    lo_b, hi_b = lax.fori_loop(0, 21, bs2, (jnp.int32(-1), jnp.int32(2 * _BIG)))
    self_bg = (keyb <= hi_b).astype(f32)

    rowp_b, rowcnt_b, crow_b = prefix_struct(self_bg)
    j_bg, o_rb, o_qb = enumerate_sel(_NUM_NEG, rowp_b, rowcnt_b, crow_b)
    key_bg = gather_at(o_rb, o_qb, keyb.astype(f32))
    ident_n = (lax.broadcasted_iota(i32, (_NUM_NEG, _NUM_NEG), 0) ==
               lax.broadcasted_iota(i32, (_NUM_NEG, _NUM_NEG), 1)).astype(f32)
    d_k = ident_n * key_bg
    krow = jnp.dot(jnp.ones((1, _NUM_NEG), f32), d_k,
                   preferred_element_type=f32, precision=_HI)
    rank_bg = jnp.sum((krow < key_bg).astype(f32), axis=1, keepdims=True)
    operm_b = (lax.broadcasted_iota(i32, (_NUM_NEG, _NUM_NEG), 1) ==
               rank_bg.astype(i32)).astype(f32)
    bgidx_ref[...] = permute(operm_b, j_bg).astype(i32)


def kernel(prp_boxes, prp_labels, tgt_boxes, tgt_labels, tgt_rel_matrix):
    out_shapes = (
        jax.ShapeDtypeStruct((_NUM_POS, 1), jnp.int32),
        jax.ShapeDtypeStruct((_NUM_POS, 1), jnp.float32),
        jax.ShapeDtypeStruct((_NUM_POS, 1), jnp.int32),
        jax.ShapeDtypeStruct((_NUM_NEG, 1), jnp.int32),
        jax.ShapeDtypeStruct((_P, _P), jnp.int32),
    )
    fgidx, fgval, fglab, bgidx, binary_rel = pl.pallas_call(
        _main_body,
        out_shape=out_shapes,
    )(tgt_boxes.astype(jnp.float32),
      prp_boxes.astype(jnp.float32).T,
      tgt_labels.astype(jnp.int32).reshape(_T, 1),
      prp_labels.astype(jnp.int32).reshape(1, _P),
      prp_labels.astype(jnp.int32).reshape(_P, 1),
      tgt_rel_matrix.astype(jnp.int32),
      jnp.asarray(_RR_CONST))

    fgidx = fgidx.reshape(-1)
    bgidx = bgidx.reshape(-1)
    rel_pairs = jnp.stack(
        [jnp.concatenate([fgidx // _P, bgidx // _P]),
         jnp.concatenate([fgidx % _P, bgidx % _P])], axis=1)
    rel_labels = jnp.concatenate(
        [fglab.reshape(-1), jnp.zeros((_NUM_NEG,), dtype=jnp.int32)])
    return rel_pairs, rel_labels, binary_rel, fgval.reshape(-1)


# speedup vs baseline: 16.1916x; 1.7747x over previous
import numpy as np
import jax
import jax.numpy as jnp
from jax import lax
from jax.experimental import pallas as pl

_P = 512
_T = 64
_NUM_POS = 256
_NUM_NEG = 768
_FG_THRES = 0.5
_HI = lax.Precision.HIGHEST
_BIG = 1 << 19


def _threefry_uniform01(seed, n):
    rotations = [np.array([13, 15, 26, 6], np.uint32),
                 np.array([17, 29, 16, 24], np.uint32)]
    k0, k1 = np.uint32(0), np.uint32(seed)
    ks = [k0, k1, np.uint32(k0 ^ k1 ^ np.uint32(0x1BD11BDA))]
    x0 = np.full(n, ks[0], np.uint32)
    x1 = (np.arange(n, dtype=np.uint32) + ks[1]).astype(np.uint32)

    def rotl(x, d):
        return ((x << np.uint32(d)) | (x >> np.uint32(32 - d))).astype(np.uint32)

    for i in range(5):
        for r in rotations[i % 2]:
            x0 = (x0 + x1).astype(np.uint32)
            x1 = rotl(x1, r)
            x1 = (x0 ^ x1).astype(np.uint32)
        x0 = (x0 + ks[(i + 1) % 3]).astype(np.uint32)
        x1 = (x1 + ks[(i + 2) % 3] + np.uint32(i + 1)).astype(np.uint32)
    bits = x0 ^ x1
    f = ((bits >> np.uint32(9)) | np.uint32(0x3F800000)).view(np.float32)
    return f - np.float32(1.0)


_RR_CACHE = None


def _rank_matrix():
    global _RR_CACHE
    if _RR_CACHE is None:
        noise = _threefry_uniform01(42, _P * _P)
        order = np.argsort(-noise, kind="stable")
        inv = np.zeros(_P * _P, np.int32)
        inv[order] = np.arange(_P * _P, dtype=np.int32)
        _RR_CACHE = inv.reshape(_P, _P)
    return _RR_CACHE


def _main_body(tb_ref, pbt_ref, tl_ref, plr_ref, plc_ref, rel_ref, rr_ref,
               fgidx_ref, fgval_ref, fglab_ref, bgidx_ref, binary_ref):
    f32 = jnp.float32
    i32 = jnp.int32
    ones_1p = jnp.ones((1, _P), f32)
    i0 = lax.broadcasted_iota(i32, (_P, _P), 0)
    i1 = lax.broadcasted_iota(i32, (_P, _P), 1)
    ut = (i0 <= i1).astype(f32)
    lst = (i1 < i0).astype(f32)
    offdiag = i0 != i1

    tb = tb_ref[...]
    pbt = pbt_ref[...]
    tl = tl_ref[...]
    plr = plr_ref[...]
    rel = rel_ref[...].astype(f32)

    tx0, ty0, tx1, ty1 = (tb[:, k:k + 1] for k in range(4))
    px0, py0, px1, py1 = (pbt[k:k + 1, :] for k in range(4))
    area_t = (tx1 - tx0) * (ty1 - ty0)
    area_p = (px1 - px0) * (py1 - py0)
    w = jnp.clip(jnp.minimum(tx1, px1) - jnp.maximum(tx0, px0), 0.0)
    h = jnp.clip(jnp.minimum(ty1, py1) - jnp.maximum(ty0, py0), 0.0)
    inter = w * h
    union = area_t + area_p - inter
    ious = inter / jnp.maximum(union, 1e-8)

    m = ((tl == plr) & (ious > _FG_THRES)).astype(f32)
    relpos = (rel > 0).astype(f32)

    inner_lab = jnp.zeros((_T, _P), f32)
    inner_iou = jnp.zeros((_T, _P), f32)
    for t in range(_T):
        inner_lab = jnp.maximum(inner_lab, rel[:, t:t + 1] * m[t:t + 1, :])
        inner_iou = jnp.maximum(inner_iou, relpos[:, t:t + 1] * ious[t:t + 1, :])

    t1 = jnp.dot(relpos, m, preferred_element_type=f32)
    b3 = lax.dot_general(m, t1, (((0,), (0,)), ((), ())),
                         preferred_element_type=f32)
    b3t = lax.dot_general(t1, m, (((0,), (0,)), ((), ())),
                          preferred_element_type=f32)
    fg_mask = (b3 > 0) & offdiag
    binary_ref[...] = ((b3 + b3t) > 0).astype(i32)

    ioust = ious.T
    fgm_f = fg_mask.astype(f32)
    validr = plr != 0
    validc = plc_ref[...] != 0
    bgm = validc & validr & offdiag & jnp.logical_not(fg_mask)

    def prefix_struct(sel_f):
        rowp = jnp.dot(sel_f, ut, preferred_element_type=f32)
        rowcnt_row = lax.dot_general(ones_1p, sel_f, (((1,), (1,)), ((), ())),
                                     preferred_element_type=f32)
        y = lax.dot_general(sel_f, ut, (((0,), (0,)), ((), ())),
                            preferred_element_type=f32)
        crow_row = jnp.sum(y, axis=0, keepdims=True)
        return rowp, rowcnt_row, crow_row

    def col_of_rowcum(sel_f):
        z = lax.dot_general(lst, sel_f, (((1,), (0,)), ((), ())),
                            preferred_element_type=f32)
        return jnp.sum(z, axis=1, keepdims=True)

    def enumerate_sel(nslots, sel_f, rowp, rowcnt_row, crow_row):
        crow_excl_row = crow_row - rowcnt_row
        svec = lax.broadcasted_iota(i32, (nslots, 1), 0).astype(f32) + 1.0
        m1 = (crow_row < svec).astype(f32)
        r_col = jnp.sum(m1, axis=1, keepdims=True)
        lane = lax.broadcasted_iota(i32, (nslots, _P), 1)
        o_r = (lane == r_col.astype(i32)).astype(f32)
        ce_at = jnp.sum(o_r * crow_excl_row, axis=1, keepdims=True)
        tgt = svec - ce_at
        selrow = jnp.dot(o_r, sel_f, preferred_element_type=f32)
        pref_rows = jnp.dot(selrow, ut, preferred_element_type=f32)
        q_col = jnp.sum((pref_rows < tgt).astype(f32), axis=1, keepdims=True)
        o_q = (lane == q_col.astype(i32)).astype(f32)
        j_col = r_col * float(_P) + q_col
        return j_col, o_r, o_q

    def row_of(col, n):
        ident = (lax.broadcasted_iota(i32, (n, n), 0) ==
                 lax.broadcasted_iota(i32, (n, n), 1)).astype(f32)
        return jnp.sum(ident * col, axis=0, keepdims=True)

    def permute_row(operm, col):
        return jnp.sum(operm * col, axis=0, keepdims=True)

    npos = jnp.sum(fgm_f)

    def common_case(_):
        return fgm_f, 1.0 - fgm_f, jnp.float32(_NUM_POS) - npos

    def rare_case(_):
        iou_pair = jnp.zeros((_P, _P), f32)
        for hh in range(_T):
            iou_pair = jnp.maximum(
                iou_pair, ioust[:, hh:hh + 1] * inner_iou[hh:hh + 1, :])
        s = jnp.where(fg_mask, iou_pair, 0.0)
        bits = lax.bitcast_convert_type(s, i32)

        def bs(_, st):
            lo, hi = st
            mid = lo + (hi - lo) // 2
            pred = jnp.sum((bits > mid).astype(i32)) < _NUM_POS
            return (jnp.where(pred, lo, mid), jnp.where(pred, mid, hi))

        _, vstar = lax.fori_loop(0, 31, bs,
                                 (jnp.int32(-1), jnp.int32(0x3F800000)))
        gtf = (bits > vstar).astype(f32)
        eqf = (bits == vstar).astype(f32)
        return gtf, eqf, jnp.float32(_NUM_POS) - jnp.sum(gtf)

    gtf, eqf, need_eq = lax.cond(npos <= _NUM_POS, common_case, rare_case, None)

    rowp_e = jnp.dot(eqf, ut, preferred_element_type=f32)
    excl_e = rowp_e - eqf + col_of_rowcum(eqf)
    self_fg = gtf + eqf * (excl_e < need_eq).astype(f32)

    rowp_f, rowcnt_f, crow_f = prefix_struct(self_fg)
    j_fg, o_r, o_q = enumerate_sel(_NUM_POS, self_fg, rowp_f, rowcnt_f, crow_f)

    iou_cols = lax.dot_general(o_r, ioust, (((1,), (0,)), ((), ())),
                               preferred_element_type=f32, precision=_HI)
    ii_cols = lax.dot_general(o_q, inner_iou, (((1,), (1,)), ((), ())),
                              preferred_element_type=f32, precision=_HI)
    fgm_at = jnp.sum(jnp.dot(o_r, fgm_f, preferred_element_type=f32) * o_q,
                     axis=1, keepdims=True)
    val_fg = jnp.where(fgm_at > 0.5,
                       jnp.max(iou_cols * ii_cols, axis=1, keepdims=True), 0.0)

    m_cols = lax.dot_general(o_r, m, (((1,), (1,)), ((), ())),
                             preferred_element_type=f32)
    il_cols = lax.dot_general(o_q, inner_lab, (((1,), (1,)), ((), ())),
                              preferred_element_type=f32)
    lab_fg = jnp.max(m_cols * il_cols, axis=1, keepdims=True)
    lab_fg = jnp.where(val_fg > 0, lab_fg, 0.0)

    vrow = row_of(val_fg, _NUM_POS)
    kc = lax.broadcasted_iota(i32, (_NUM_POS, 1), 0)
    kr = lax.broadcasted_iota(i32, (1, _NUM_POS), 1)
    beats = (vrow > val_fg) | ((vrow == val_fg) & (kr < kc))
    rank_fg = jnp.sum(beats.astype(f32), axis=1, keepdims=True)
    operm = (lax.broadcasted_iota(i32, (_NUM_POS, _NUM_POS), 1) ==
             rank_fg.astype(i32)).astype(f32)

    fgval_ref[...] = permute_row(operm, val_fg)
    fgidx_ref[...] = permute_row(operm, j_fg).astype(i32)
    fglab_ref[...] = permute_row(operm, lab_fg).astype(i32)

    jmat = i0 * _P + i1
    keyb = jnp.where(bgm, rr_ref[...], _BIG + jmat)

    def bs2(_, st):
        lo, hi = st
        mid = lo + (hi - lo) // 2
        pred = jnp.sum((keyb <= mid).astype(i32)) >= _NUM_NEG
        return (jnp.where(pred, lo, mid), jnp.where(pred, mid, hi))

    _, rstar = lax.fori_loop(0, 21, bs2, (jnp.int32(-1), jnp.int32(2 * _BIG)))
    self_bg = (keyb <= rstar).astype(f32)

    rowp_b, rowcnt_b, crow_b = prefix_struct(self_bg)
    j_bg, o_rb, o_qb = enumerate_sel(_NUM_NEG, self_bg,
                                     rowp_b, rowcnt_b, crow_b)
    key_rows = jnp.dot(o_rb, keyb.astype(f32),
                       preferred_element_type=f32, precision=_HI)
    key_bg = jnp.sum(key_rows * o_qb, axis=1, keepdims=True)
    krow = row_of(key_bg, _NUM_NEG)
    rank_bg = jnp.sum((krow < key_bg).astype(f32), axis=1, keepdims=True)
    operm_b = (lax.broadcasted_iota(i32, (_NUM_NEG, _NUM_NEG), 1) ==
               rank_bg.astype(i32)).astype(f32)
    bgidx_ref[...] = permute_row(operm_b, j_bg).astype(i32)


def kernel(prp_boxes, prp_labels, tgt_boxes, tgt_labels, tgt_rel_matrix):
    out_shapes = (
        jax.ShapeDtypeStruct((1, _NUM_POS), jnp.int32),
        jax.ShapeDtypeStruct((1, _NUM_POS), jnp.float32),
        jax.ShapeDtypeStruct((1, _NUM_POS), jnp.int32),
        jax.ShapeDtypeStruct((1, _NUM_NEG), jnp.int32),
        jax.ShapeDtypeStruct((_P, _P), jnp.int32),
    )
    fgidx, fgval, fglab, bgidx, binary_rel = pl.pallas_call(
        _main_body,
        out_shape=out_shapes,
    )(tgt_boxes.astype(jnp.float32),
      prp_boxes.astype(jnp.float32).T,
      tgt_labels.astype(jnp.int32).reshape(_T, 1),
      prp_labels.astype(jnp.int32).reshape(1, _P),
      prp_labels.astype(jnp.int32).reshape(_P, 1),
      tgt_rel_matrix.astype(jnp.int32),
      jnp.asarray(_rank_matrix()))

    fgidx = fgidx.reshape(-1)
    bgidx = bgidx.reshape(-1)
    rel_pairs = jnp.stack(
        [jnp.concatenate([fgidx // _P, bgidx // _P]),
         jnp.concatenate([fgidx % _P, bgidx % _P])], axis=1)
    rel_labels = jnp.concatenate(
        [fglab.reshape(-1), jnp.zeros((_NUM_NEG,), dtype=jnp.int32)])
    return rel_pairs, rel_labels, binary_rel, fgval.reshape(-1)


# speedup vs baseline: 17.2924x; 1.0680x over previous
import numpy as np
import jax
import jax.numpy as jnp
from jax import lax
from jax.experimental import pallas as pl

_P = 512
_T = 64
_NUM_POS = 256
_NUM_NEG = 768
_FG_THRES = 0.5
_HI = lax.Precision.HIGHEST
_BIG = 1 << 19


def _threefry_uniform01(seed, n):
    rotations = [np.array([13, 15, 26, 6], np.uint32),
                 np.array([17, 29, 16, 24], np.uint32)]
    k0, k1 = np.uint32(0), np.uint32(seed)
    ks = [k0, k1, np.uint32(k0 ^ k1 ^ np.uint32(0x1BD11BDA))]
    x0 = np.full(n, ks[0], np.uint32)
    x1 = (np.arange(n, dtype=np.uint32) + ks[1]).astype(np.uint32)

    def rotl(x, d):
        return ((x << np.uint32(d)) | (x >> np.uint32(32 - d))).astype(np.uint32)

    for i in range(5):
        for r in rotations[i % 2]:
            x0 = (x0 + x1).astype(np.uint32)
            x1 = rotl(x1, r)
            x1 = (x0 ^ x1).astype(np.uint32)
        x0 = (x0 + ks[(i + 1) % 3]).astype(np.uint32)
        x1 = (x1 + ks[(i + 2) % 3] + np.uint32(i + 1)).astype(np.uint32)
    bits = x0 ^ x1
    f = ((bits >> np.uint32(9)) | np.uint32(0x3F800000)).view(np.float32)
    return f - np.float32(1.0)


_RR_CACHE = None


def _rank_matrix():
    global _RR_CACHE
    if _RR_CACHE is None:
        noise = _threefry_uniform01(42, _P * _P)
        order = np.argsort(-noise, kind="stable")
        inv = np.zeros(_P * _P, np.int32)
        inv[order] = np.arange(_P * _P, dtype=np.int32)
        _RR_CACHE = inv.reshape(_P, _P)
    return _RR_CACHE


def _main_body(tb_ref, pbt_ref, tl_ref, plr_ref, plc_ref, rel_ref, rr_ref,
               fgidx_ref, fgval_ref, fglab_ref, bgidx_ref, binary_ref):
    f32 = jnp.float32
    i32 = jnp.int32
    ones_1p = jnp.ones((1, _P), f32)
    i0 = lax.broadcasted_iota(i32, (_P, _P), 0)
    i1 = lax.broadcasted_iota(i32, (_P, _P), 1)
    bf16 = jnp.bfloat16
    ut = (i0 <= i1).astype(bf16)
    lst = (i1 < i0).astype(bf16)
    offdiag = i0 != i1

    tb = tb_ref[...]
    pbt = pbt_ref[...]
    tl = tl_ref[...]
    plr = plr_ref[...]
    rel = rel_ref[...].astype(f32)

    tx0, ty0, tx1, ty1 = (tb[:, k:k + 1] for k in range(4))
    px0, py0, px1, py1 = (pbt[k:k + 1, :] for k in range(4))
    area_t = (tx1 - tx0) * (ty1 - ty0)
    area_p = (px1 - px0) * (py1 - py0)
    w = jnp.clip(jnp.minimum(tx1, px1) - jnp.maximum(tx0, px0), 0.0)
    h = jnp.clip(jnp.minimum(ty1, py1) - jnp.maximum(ty0, py0), 0.0)
    inter = w * h
    union = area_t + area_p - inter
    ious = inter / jnp.maximum(union, 1e-8)

    m = ((tl == plr) & (ious > _FG_THRES)).astype(f32)
    relpos = (rel > 0).astype(f32)

    inner_lab = jnp.zeros((_T, _P), f32)
    inner_iou = jnp.zeros((_T, _P), f32)
    for t in range(_T):
        inner_lab = jnp.maximum(inner_lab, rel[:, t:t + 1] * m[t:t + 1, :])
        inner_iou = jnp.maximum(inner_iou, relpos[:, t:t + 1] * ious[t:t + 1, :])

    m_bf = m.astype(bf16)
    t1 = jnp.dot(relpos.astype(bf16), m_bf, preferred_element_type=f32)
    t1_bf = t1.astype(bf16)
    b3 = lax.dot_general(m_bf, t1_bf, (((0,), (0,)), ((), ())),
                         preferred_element_type=f32)
    b3t = lax.dot_general(t1_bf, m_bf, (((0,), (0,)), ((), ())),
                          preferred_element_type=f32)
    fg_mask = (b3 > 0) & offdiag
    binary_ref[...] = ((b3 + b3t) > 0).astype(i32)

    ioust = ious.T
    fgm_f = fg_mask.astype(f32)
    validr = plr != 0
    validc = plc_ref[...] != 0
    bgm = validc & validr & offdiag & jnp.logical_not(fg_mask)

    ones_1p_bf = jnp.ones((1, _P), bf16)

    def prefix_struct(sel_b):
        rowp = jnp.dot(sel_b, ut, preferred_element_type=f32)
        rowcnt_row = lax.dot_general(ones_1p_bf, sel_b, (((1,), (1,)), ((), ())),
                                     preferred_element_type=f32)
        y = lax.dot_general(sel_b, ut, (((0,), (0,)), ((), ())),
                            preferred_element_type=f32)
        crow_row = jnp.sum(y, axis=0, keepdims=True)
        return rowp, rowcnt_row, crow_row

    def col_of_rowcum(sel_b):
        z = lax.dot_general(lst, sel_b, (((1,), (0,)), ((), ())),
                            preferred_element_type=f32)
        return jnp.sum(z, axis=1, keepdims=True)

    def enumerate_sel(nslots, sel_b, rowp, rowcnt_row, crow_row):
        crow_excl_row = crow_row - rowcnt_row
        svec = lax.broadcasted_iota(i32, (nslots, 1), 0).astype(f32) + 1.0
        m1 = (crow_row < svec).astype(f32)
        r_col = jnp.sum(m1, axis=1, keepdims=True)
        lane = lax.broadcasted_iota(i32, (nslots, _P), 1)
        o_r = (lane == r_col.astype(i32)).astype(f32)
        ce_at = jnp.sum(o_r * crow_excl_row, axis=1, keepdims=True)
        tgt = svec - ce_at
        selrow = jnp.dot(o_r.astype(bf16), sel_b, preferred_element_type=f32)
        pref_rows = jnp.dot(selrow.astype(bf16), ut, preferred_element_type=f32)
        q_col = jnp.sum((pref_rows < tgt).astype(f32), axis=1, keepdims=True)
        o_q = (lane == q_col.astype(i32)).astype(f32)
        j_col = r_col * float(_P) + q_col
        return j_col, o_r, o_q

    def row_of(col, n):
        ident = (lax.broadcasted_iota(i32, (n, n), 0) ==
                 lax.broadcasted_iota(i32, (n, n), 1)).astype(f32)
        return jnp.sum(ident * col, axis=0, keepdims=True)

    def permute_row(operm, col):
        return jnp.sum(operm * col, axis=0, keepdims=True)

    npos = jnp.sum(fgm_f)

    def common_case(_):
        return fgm_f, 1.0 - fgm_f, jnp.float32(_NUM_POS) - npos

    def rare_case(_):
        iou_pair = jnp.zeros((_P, _P), f32)
        for hh in range(_T):
            iou_pair = jnp.maximum(
                iou_pair, ioust[:, hh:hh + 1] * inner_iou[hh:hh + 1, :])
        s = jnp.where(fg_mask, iou_pair, 0.0)
        bits = lax.bitcast_convert_type(s, i32)

        def bs(_, st):
            lo, hi = st
            mid = lo + (hi - lo) // 2
            pred = jnp.sum((bits > mid).astype(i32)) < _NUM_POS
            return (jnp.where(pred, lo, mid), jnp.where(pred, mid, hi))

        _, vstar = lax.fori_loop(0, 31, bs,
                                 (jnp.int32(-1), jnp.int32(0x3F800000)))
        gtf = (bits > vstar).astype(f32)
        eqf = (bits == vstar).astype(f32)
        return gtf, eqf, jnp.float32(_NUM_POS) - jnp.sum(gtf)

    gtf, eqf, need_eq = lax.cond(npos <= _NUM_POS, common_case, rare_case, None)

    eqf_bf = eqf.astype(bf16)
    rowp_e = jnp.dot(eqf_bf, ut, preferred_element_type=f32)
    excl_e = rowp_e - eqf + col_of_rowcum(eqf_bf)
    self_fg = gtf + eqf * (excl_e < need_eq).astype(f32)

    self_fg_bf = self_fg.astype(bf16)
    rowp_f, rowcnt_f, crow_f = prefix_struct(self_fg_bf)
    j_fg, o_r, o_q = enumerate_sel(_NUM_POS, self_fg_bf,
                                   rowp_f, rowcnt_f, crow_f)

    iou_cols = lax.dot_general(o_r, ioust, (((1,), (0,)), ((), ())),
                               preferred_element_type=f32, precision=_HI)
    ii_cols = lax.dot_general(o_q, inner_iou, (((1,), (1,)), ((), ())),
                              preferred_element_type=f32, precision=_HI)
    fgm_at = jnp.sum(jnp.dot(o_r.astype(bf16), fgm_f.astype(bf16),
                             preferred_element_type=f32) * o_q,
                     axis=1, keepdims=True)
    val_fg = jnp.where(fgm_at > 0.5,
                       jnp.max(iou_cols * ii_cols, axis=1, keepdims=True), 0.0)

    m_cols = lax.dot_general(o_r.astype(bf16), m_bf, (((1,), (1,)), ((), ())),
                             preferred_element_type=f32)
    il_cols = lax.dot_general(o_q.astype(bf16), inner_lab.astype(bf16),
                              (((1,), (1,)), ((), ())),
                              preferred_element_type=f32)
    lab_fg = jnp.max(m_cols * il_cols, axis=1, keepdims=True)
    lab_fg = jnp.where(val_fg > 0, lab_fg, 0.0)

    vrow = row_of(val_fg, _NUM_POS)
    kc = lax.broadcasted_iota(i32, (_NUM_POS, 1), 0)
    kr = lax.broadcasted_iota(i32, (1, _NUM_POS), 1)
    beats = (vrow > val_fg) | ((vrow == val_fg) & (kr < kc))
    rank_fg = jnp.sum(beats.astype(f32), axis=1, keepdims=True)
    operm = (lax.broadcasted_iota(i32, (_NUM_POS, _NUM_POS), 1) ==
             rank_fg.astype(i32)).astype(f32)

    fgval_ref[...] = permute_row(operm, val_fg)
    fgidx_ref[...] = permute_row(operm, j_fg).astype(i32)
    fglab_ref[...] = permute_row(operm, lab_fg).astype(i32)

    jmat = i0 * _P + i1
    keyb = jnp.where(bgm, rr_ref[...], _BIG + jmat)

    def bs2(_, st):
        lo, hi = st
        mid = lo + (hi - lo) // 2
        pred = jnp.sum((keyb <= mid).astype(i32)) >= _NUM_NEG
        return (jnp.where(pred, lo, mid), jnp.where(pred, mid, hi))

    _, rstar = lax.fori_loop(0, 21, bs2, (jnp.int32(-1), jnp.int32(2 * _BIG)))
    self_bg = (keyb <= rstar).astype(f32)

    self_bg_bf = self_bg.astype(bf16)
    rowp_b, rowcnt_b, crow_b = prefix_struct(self_bg_bf)
    j_bg, o_rb, o_qb = enumerate_sel(_NUM_NEG, self_bg_bf,
                                     rowp_b, rowcnt_b, crow_b)
    o_rb_bf = o_rb.astype(bf16)
    kb0 = (keyb & 255).astype(bf16)
    kb1 = ((keyb >> 8) & 255).astype(bf16)
    kb2 = (keyb >> 16).astype(bf16)
    g0 = jnp.dot(o_rb_bf, kb0, preferred_element_type=f32)
    g1 = jnp.dot(o_rb_bf, kb1, preferred_element_type=f32)
    g2 = jnp.dot(o_rb_bf, kb2, preferred_element_type=f32)
    key_bg = jnp.sum((g2 * 65536.0 + g1 * 256.0 + g0) * o_qb,
                     axis=1, keepdims=True)
    krow = row_of(key_bg, _NUM_NEG)
    rank_bg = jnp.sum((krow < key_bg).astype(f32), axis=1, keepdims=True)
    operm_b = (lax.broadcasted_iota(i32, (_NUM_NEG, _NUM_NEG), 1) ==
               rank_bg.astype(i32)).astype(f32)
    bgidx_ref[...] = permute_row(operm_b, j_bg).astype(i32)


def kernel(prp_boxes, prp_labels, tgt_boxes, tgt_labels, tgt_rel_matrix):
    out_shapes = (
        jax.ShapeDtypeStruct((1, _NUM_POS), jnp.int32),
        jax.ShapeDtypeStruct((1, _NUM_POS), jnp.float32),
        jax.ShapeDtypeStruct((1, _NUM_POS), jnp.int32),
        jax.ShapeDtypeStruct((1, _NUM_NEG), jnp.int32),
        jax.ShapeDtypeStruct((_P, _P), jnp.int32),
    )
    fgidx, fgval, fglab, bgidx, binary_rel = pl.pallas_call(
        _main_body,
        out_shape=out_shapes,
    )(tgt_boxes.astype(jnp.float32),
      prp_boxes.astype(jnp.float32).T,
      tgt_labels.astype(jnp.int32).reshape(_T, 1),
      prp_labels.astype(jnp.int32).reshape(1, _P),
      prp_labels.astype(jnp.int32).reshape(_P, 1),
      tgt_rel_matrix.astype(jnp.int32),
      jnp.asarray(_rank_matrix()))

    fgidx = fgidx.reshape(-1)
    bgidx = bgidx.reshape(-1)
    rel_pairs = jnp.stack(
        [jnp.concatenate([fgidx // _P, bgidx // _P]),
         jnp.concatenate([fgidx % _P, bgidx % _P])], axis=1)
    rel_labels = jnp.concatenate(
        [fglab.reshape(-1), jnp.zeros((_NUM_NEG,), dtype=jnp.int32)])
    return rel_pairs, rel_labels, binary_rel, fgval.reshape(-1)
